# baseline (speedup 1.0000x reference)
import functools

import jax
import jax.numpy as jnp
from jax import lax
from jax.experimental import pallas as pl
from jax.experimental.pallas import tpu as pltpu
from jax.experimental.pallas import tpu_sc as plsc

_N = 10000
_E = 320000
_H = 128
_HH = _H // 2
_NG = 64
_MD = 16

_W = 128
_NROW = _E // _W
_STRIPE = 624
_SCH = 104
_TAIL = _N - 16 * _STRIPE


def _split_emb_body(x_ref, w_ref, b_ref, o_ref):
    o_ref[0] = (
        jnp.dot(x_ref[...], w_ref[0], preferred_element_type=jnp.float32)
        + b_ref[0]
    )


def _node_emb(x, w, b):
    w2 = jnp.stack([w[:, :_HH], w[:, _HH:]])
    b2 = jnp.stack([b[:_HH], b[_HH:]]).reshape(2, 1, _HH)
    return pl.pallas_call(
        _split_emb_body,
        grid=(2,),
        in_specs=[
            pl.BlockSpec((_N, x.shape[1]), lambda j: (0, 0)),
            pl.BlockSpec((1, x.shape[1], _HH), lambda j: (j, 0, 0)),
            pl.BlockSpec((1, 1, _HH), lambda j: (j, 0, 0)),
        ],
        out_specs=pl.BlockSpec((1, _N, _HH), lambda j: (j, 0, 0)),
        out_shape=jax.ShapeDtypeStruct((2, _N, _HH), jnp.float32),
    )(x, w2, b2)


def _edge_emb(ea, w, b):
    blk = 8000
    de = ea.shape[1]
    w2 = jnp.stack([w[:, :_HH], w[:, _HH:]])
    b2 = jnp.stack([b[:_HH], b[_HH:]]).reshape(2, 1, _HH)
    return pl.pallas_call(
        _split_emb_body,
        grid=(2, _E // blk),
        in_specs=[
            pl.BlockSpec((blk, de), lambda j, i: (i, 0)),
            pl.BlockSpec((1, de, _HH), lambda j, i: (j, 0, 0)),
            pl.BlockSpec((1, 1, _HH), lambda j, i: (j, 0, 0)),
        ],
        out_specs=pl.BlockSpec((1, blk, _HH), lambda j, i: (j, i, 0)),
        out_shape=jax.ShapeDtypeStruct((2, _E, _HH), jnp.float32),
    )(ea, w2, b2)


def _mlp_bn_body(h_ref, agg_ref, w1_ref, b1_ref, w2_ref, b2_ref, g_ref,
                 bb_ref, o_ref):
    z = h_ref[...] + jnp.concatenate([agg_ref[0], agg_ref[1]], axis=1)
    z1 = jnp.maximum(
        jnp.dot(z, w1_ref[...], preferred_element_type=jnp.float32)
        + b1_ref[...], 0.0)
    z2 = (jnp.dot(z1, w2_ref[...], preferred_element_type=jnp.float32)
          + b2_ref[...])
    mean = jnp.mean(z2, axis=0, keepdims=True)
    var = jnp.mean((z2 - mean) ** 2, axis=0, keepdims=True)
    zn = (z2 - mean) * lax.rsqrt(var + 1e-5) * g_ref[...] + bb_ref[...]
    o_ref[...] = jnp.maximum(zn, 0.0)


def _mlp_bn(h, agg, lp):
    return pl.pallas_call(
        _mlp_bn_body,
        out_shape=jax.ShapeDtypeStruct((_N, _H), jnp.float32),
    )(h, agg, lp["W1"], lp["b1"].reshape(1, -1), lp["W2"],
      lp["b2"].reshape(1, -1), lp["bn_g"].reshape(1, -1),
      lp["bn_b"].reshape(1, -1))


def _pool_head_body(h_ref, b_ref, wc_ref, bc_ref, wu_ref, bu_ref, wfu_ref,
                    wfc_ref, bf_ref, o_ref):
    gids = lax.broadcasted_iota(jnp.int32, (_N, _NG), 1)
    onehot = (b_ref[...] == gids).astype(jnp.float32)
    sums = lax.dot_general(onehot, h_ref[...], (((0,), (0,)), ((), ())),
                           preferred_element_type=jnp.float32)
    counts = jnp.sum(onehot, axis=0)[:, None]
    gx = sums / jnp.maximum(counts, 1.0)
    eu = jnp.dot(gx, wu_ref[...], preferred_element_type=jnp.float32) + bu_ref[...]
    ec = jnp.dot(gx, wc_ref[...], preferred_element_type=jnp.float32) + bc_ref[...]
    o_ref[...] = (
        jnp.dot(eu, wfu_ref[...], preferred_element_type=jnp.float32)
        + jnp.dot(ec, wfc_ref[...], preferred_element_type=jnp.float32)
        + bf_ref[...])


def _pool_head(h, batch, params):
    wf = params["final"]["W"]
    nc = wf.shape[1]
    return pl.pallas_call(
        _pool_head_body,
        out_shape=jax.ShapeDtypeStruct((_NG, nc), jnp.float32),
    )(h, batch.reshape(_N, 1),
      params["lin_common"]["W"], params["lin_common"]["b"].reshape(1, -1),
      params["lin_uncommon"]["W"], params["lin_uncommon"]["b"].reshape(1, -1),
      wf[:_MD], wf[_MD:], params["final"]["b"].reshape(1, -1))


def _sc_message(hp, ep, src2, dst2):
    mesh = plsc.VectorSubcoreMesh(core_axis_name="c", subcore_axis_name="s")

    @functools.partial(
        pl.kernel,
        mesh=mesh,
        compiler_params=pltpu.CompilerParams(use_tc_tiling_on_sc=False),
        out_type=jax.ShapeDtypeStruct((2, _N, _HH), jnp.float32),
        scratch_types=[
            pltpu.VMEM((2, 1, _W), jnp.int32),
            pltpu.VMEM((2, 2, _W), jnp.int32),
            pltpu.VMEM((2, _W, _HH), jnp.float32),
            pltpu.VMEM((2, _W, _HH), jnp.float32),
            pltpu.VMEM((2, _W, _HH), jnp.float32),
            pltpu.VMEM_SHARED((_N, _HH), jnp.float32),
            pltpu.SemaphoreType.DMA,
            pltpu.SemaphoreType.DMA,
            pltpu.SemaphoreType.DMA,
            pltpu.SemaphoreType.DMA,
            pltpu.SemaphoreType.DMA,
            pltpu.SemaphoreType.DMA,
            pltpu.SemaphoreType.DMA,
            pltpu.SemaphoreType.DMA,
        ],
    )
    def k(hp_hbm, ep_hbm, src_hbm, dst_hbm, out_hbm, src_v, dst_v, rows_v,
          e_v, m_v, agg_sh, gs0, gs1, es0, es1, ss0, ss1, is0, is1):
        c = lax.axis_index("c")
        s = lax.axis_index("s")
        gsem = (gs0, gs1)
        esem = (es0, es1)
        ssem = (ss0, ss1)
        isem = (is0, is1)
        hv = hp_hbm.at[c]
        ev_hbm = ep_hbm.at[c]

        zero16 = jnp.zeros((16,), jnp.float32)

        def zrow(r, carry):
            for j in range(_HH // 16):
                m_v[0, r, pl.ds(j * 16, 16)] = zero16
            return carry

        lax.fori_loop(0, _SCH, zrow, 0)
        r0 = s * _STRIPE
        for t in range(_STRIPE // _SCH):
            pltpu.sync_copy(m_v.at[0, pl.ds(0, _SCH)],
                            agg_sh.at[pl.ds(r0 + t * _SCH, _SCH)])

        @pl.when(s == 15)
        def _zero_tail():
            pltpu.sync_copy(m_v.at[0, pl.ds(0, _TAIL)],
                            agg_sh.at[pl.ds(16 * _STRIPE, _TAIL)])

        plsc.subcore_barrier()

        lo = 156 * s + 2 * jnp.maximum(s - 14, 0)
        cnt = 156 + 2 * (s >= 14).astype(jnp.int32)
        hi = lo + cnt

        def launch(S, r):
            pltpu.async_copy(hv.at[src_v.at[S, 0]], rows_v.at[S], gsem[S])
            pltpu.async_copy(ev_hbm.at[pl.ds(r * _W, _W)], e_v.at[S],
                             esem[S])

        def drain(sem, dst, dummy):
            pltpu.make_async_copy(dummy.at[pl.ds(0, dst.shape[0])], dst,
                                  sem).wait()

        for S in (0, 1):
            pltpu.sync_copy(src_hbm.at[pl.ds(lo + S, 1)], src_v.at[S])
            pltpu.sync_copy(dst_hbm.at[pl.ds(lo + S, 1)],
                            dst_v.at[S, pl.ds(0, 1)])
            launch(S, lo + S)

        def body(kk, carry):
            q = kk & 1
            for S in (0, 1):
                i = lo + 2 * kk + S
                drain(gsem[S], rows_v.at[S], hv)

                @pl.when(kk >= 1)
                def _drain_sc():
                    drain(ssem[S], m_v.at[S], hv)

                @pl.when(i + 2 < hi)
                def _idx_prefetch():
                    pltpu.async_copy(src_hbm.at[pl.ds(i + 2, 1)],
                                     src_v.at[S], isem[S])
                    pltpu.async_copy(dst_hbm.at[pl.ds(i + 2, 1)],
                                     dst_v.at[S, pl.ds(1 - q, 1)], isem[S])

                drain(esem[S], e_v.at[S], hv)

                def crow(rr, cy):
                    for j in range(_HH // 16):
                        sl = pl.ds(j * 16, 16)
                        m_v[S, rr, sl] = jnp.maximum(
                            rows_v[S, rr, sl] + e_v[S, rr, sl], 0.0)
                    return cy

                lax.fori_loop(0, _W, crow, 0)
                pltpu.async_copy(m_v.at[S], agg_sh.at[dst_v.at[S, q]],
                                 ssem[S], add=True)

                @pl.when(i + 2 < hi)
                def _next_launch():
                    drain(isem[S], src_v.at[S], src_hbm)
                    drain(isem[S], dst_v.at[S, pl.ds(1 - q, 1)], src_hbm)
                    launch(S, i + 2)
            return carry

        lax.fori_loop(0, cnt // 2, body, 0)
        for S in (0, 1):
            drain(ssem[S], m_v.at[S], hv)
        plsc.subcore_barrier()

        for t in range(_STRIPE // _SCH):
            rr = r0 + t * _SCH
            pltpu.sync_copy(agg_sh.at[pl.ds(rr, _SCH)],
                            rows_v.at[0, pl.ds(0, _SCH)])
            pltpu.sync_copy(rows_v.at[0, pl.ds(0, _SCH)],
                            out_hbm.at[c, pl.ds(rr, _SCH)])

        @pl.when(s == 15)
        def _write_tail():
            pltpu.sync_copy(agg_sh.at[pl.ds(16 * _STRIPE, _TAIL)],
                            rows_v.at[0, pl.ds(0, _TAIL)])
            pltpu.sync_copy(rows_v.at[0, pl.ds(0, _TAIL)],
                            out_hbm.at[c, pl.ds(16 * _STRIPE, _TAIL)])

    return k(hp, ep, src2, dst2)


def kernel(x, edge_attr, params, edge_index, batch):
    src2 = edge_index[0].reshape(_NROW, _W)
    dst2 = edge_index[1].reshape(_NROW, _W)
    hp = _node_emb(x, params["node_emb"]["W"], params["node_emb"]["b"])
    h = jnp.concatenate([hp[0], hp[1]], axis=1)
    ep = _edge_emb(edge_attr, params["edge_emb"]["W"], params["edge_emb"]["b"])
    for li, lp in enumerate(params["layers"]):
        agg = _sc_message(hp, ep, src2, dst2)
        h = _mlp_bn(h, agg, lp)
        if li + 1 < len(params["layers"]):
            hp = jnp.stack([h[:, :_HH], h[:, _HH:]])
    return _pool_head(h, batch, params)


# speedup vs baseline: 1.3710x; 1.3710x over previous
import functools

import jax
import jax.numpy as jnp
from jax import lax
from jax.experimental import pallas as pl
from jax.experimental.pallas import tpu as pltpu
from jax.experimental.pallas import tpu_sc as plsc

_N = 10000
_E = 320000
_H = 128
_HH = _H // 2
_NG = 64
_MD = 16

_W = 128
_NROW = _E // _W
_STRIPE = 624
_SCH = 104
_TAIL = _N - 16 * _STRIPE


def _split_emb_body(x_ref, w_ref, b_ref, o_ref):
    o_ref[0] = (
        jnp.dot(x_ref[...], w_ref[0], preferred_element_type=jnp.float32)
        + b_ref[0]
    )


def _node_emb(x, w, b):
    w2 = jnp.stack([w[:, :_HH], w[:, _HH:]])
    b2 = jnp.stack([b[:_HH], b[_HH:]]).reshape(2, 1, _HH)
    return pl.pallas_call(
        _split_emb_body,
        grid=(2,),
        in_specs=[
            pl.BlockSpec((_N, x.shape[1]), lambda j: (0, 0)),
            pl.BlockSpec((1, x.shape[1], _HH), lambda j: (j, 0, 0)),
            pl.BlockSpec((1, 1, _HH), lambda j: (j, 0, 0)),
        ],
        out_specs=pl.BlockSpec((1, _N, _HH), lambda j: (j, 0, 0)),
        out_shape=jax.ShapeDtypeStruct((2, _N, _HH), jnp.float32),
    )(x, w2, b2)


def _edge_emb_body(a_ref, w_ref, b_ref, o_ref):
    a = a_ref[...]
    de = a.shape[1] // 2
    w = w_ref[0]
    b = b_ref[0]
    el = jnp.dot(a[:, :de], w, preferred_element_type=jnp.float32) + b
    er = jnp.dot(a[:, de:], w, preferred_element_type=jnp.float32) + b
    o_ref[0] = jnp.concatenate([el, er], axis=1)


def _edge_emb(ea2, w, b):
    blk = 8000
    de2 = ea2.shape[1]
    w2 = jnp.stack([w[:, :_HH], w[:, _HH:]])
    b2 = jnp.stack([b[:_HH], b[_HH:]]).reshape(2, 1, _HH)
    return pl.pallas_call(
        _edge_emb_body,
        grid=(2, _E // 2 // blk),
        in_specs=[
            pl.BlockSpec((blk, de2), lambda j, i: (i, 0)),
            pl.BlockSpec((1, de2 // 2, _HH), lambda j, i: (j, 0, 0)),
            pl.BlockSpec((1, 1, _HH), lambda j, i: (j, 0, 0)),
        ],
        out_specs=pl.BlockSpec((1, blk, _H), lambda j, i: (j, i, 0)),
        out_shape=jax.ShapeDtypeStruct((2, _E // 2, _H), jnp.float32),
    )(ea2, w2, b2)


def _mlp_bn_body(h_ref, agg_ref, w1_ref, b1_ref, w2_ref, b2_ref, g_ref,
                 bb_ref, o_ref):
    z = h_ref[...] + jnp.concatenate([agg_ref[0], agg_ref[1]], axis=1)
    z1 = jnp.maximum(
        jnp.dot(z, w1_ref[...], preferred_element_type=jnp.float32)
        + b1_ref[...], 0.0)
    z2 = (jnp.dot(z1, w2_ref[...], preferred_element_type=jnp.float32)
          + b2_ref[...])
    mean = jnp.mean(z2, axis=0, keepdims=True)
    var = jnp.mean((z2 - mean) ** 2, axis=0, keepdims=True)
    zn = (z2 - mean) * lax.rsqrt(var + 1e-5) * g_ref[...] + bb_ref[...]
    o_ref[...] = jnp.maximum(zn, 0.0)


def _mlp_bn(h, agg, lp):
    return pl.pallas_call(
        _mlp_bn_body,
        out_shape=jax.ShapeDtypeStruct((_N, _H), jnp.float32),
    )(h, agg, lp["W1"], lp["b1"].reshape(1, -1), lp["W2"],
      lp["b2"].reshape(1, -1), lp["bn_g"].reshape(1, -1),
      lp["bn_b"].reshape(1, -1))


def _pool_head_body(h_ref, b_ref, wc_ref, bc_ref, wu_ref, bu_ref, wfu_ref,
                    wfc_ref, bf_ref, o_ref):
    gids = lax.broadcasted_iota(jnp.int32, (_N, _NG), 1)
    onehot = (b_ref[...] == gids).astype(jnp.float32)
    sums = lax.dot_general(onehot, h_ref[...], (((0,), (0,)), ((), ())),
                           preferred_element_type=jnp.float32)
    counts = jnp.sum(onehot, axis=0)[:, None]
    gx = sums / jnp.maximum(counts, 1.0)
    eu = jnp.dot(gx, wu_ref[...], preferred_element_type=jnp.float32) + bu_ref[...]
    ec = jnp.dot(gx, wc_ref[...], preferred_element_type=jnp.float32) + bc_ref[...]
    o_ref[...] = (
        jnp.dot(eu, wfu_ref[...], preferred_element_type=jnp.float32)
        + jnp.dot(ec, wfc_ref[...], preferred_element_type=jnp.float32)
        + bf_ref[...])


def _pool_head(h, batch, params):
    wf = params["final"]["W"]
    nc = wf.shape[1]
    return pl.pallas_call(
        _pool_head_body,
        out_shape=jax.ShapeDtypeStruct((_NG, nc), jnp.float32),
    )(h, batch.reshape(_N, 1),
      params["lin_common"]["W"], params["lin_common"]["b"].reshape(1, -1),
      params["lin_uncommon"]["W"], params["lin_uncommon"]["b"].reshape(1, -1),
      wf[:_MD], wf[_MD:], params["final"]["b"].reshape(1, -1))


def _sc_message(hp, ep, src2, dst2):
    mesh = plsc.VectorSubcoreMesh(core_axis_name="c", subcore_axis_name="s")

    @functools.partial(
        pl.kernel,
        mesh=mesh,
        compiler_params=pltpu.CompilerParams(use_tc_tiling_on_sc=False),
        out_type=jax.ShapeDtypeStruct((2, _N, _HH), jnp.float32),
        scratch_types=[
            pltpu.VMEM((2, 1, _W), jnp.int32),
            pltpu.VMEM((2, 2, _W), jnp.int32),
            pltpu.VMEM((2, _W, _HH), jnp.float32),
            pltpu.VMEM((2, _W // 2, _H), jnp.float32),
            pltpu.VMEM((2, _W, _HH), jnp.float32),
            pltpu.VMEM_SHARED((_N, _HH), jnp.float32),
            pltpu.SemaphoreType.DMA,
            pltpu.SemaphoreType.DMA,
            pltpu.SemaphoreType.DMA,
            pltpu.SemaphoreType.DMA,
            pltpu.SemaphoreType.DMA,
            pltpu.SemaphoreType.DMA,
            pltpu.SemaphoreType.DMA,
            pltpu.SemaphoreType.DMA,
        ],
    )
    def k(hp_hbm, ep_hbm, src_hbm, dst_hbm, out_hbm, src_v, dst_v, rows_v,
          e_v, m_v, agg_sh, gs0, gs1, es0, es1, ss0, ss1, is0, is1):
        c = lax.axis_index("c")
        s = lax.axis_index("s")
        gsem = (gs0, gs1)
        esem = (es0, es1)
        ssem = (ss0, ss1)
        isem = (is0, is1)
        hv = hp_hbm.at[c]
        ev_hbm = ep_hbm.at[c]

        zero16 = jnp.zeros((16,), jnp.float32)

        def zrow(r, carry):
            for j in range(_HH // 16):
                m_v[0, r, pl.ds(j * 16, 16)] = zero16
            return carry

        lax.fori_loop(0, _SCH, zrow, 0)
        r0 = s * _STRIPE
        for t in range(_STRIPE // _SCH):
            pltpu.sync_copy(m_v.at[0, pl.ds(0, _SCH)],
                            agg_sh.at[pl.ds(r0 + t * _SCH, _SCH)])

        @pl.when(s == 15)
        def _zero_tail():
            pltpu.sync_copy(m_v.at[0, pl.ds(0, _TAIL)],
                            agg_sh.at[pl.ds(16 * _STRIPE, _TAIL)])

        plsc.subcore_barrier()

        lo = 156 * s + 2 * jnp.maximum(s - 14, 0)
        cnt = 156 + 2 * (s >= 14).astype(jnp.int32)
        hi = lo + cnt

        def launch(S, r):
            pltpu.async_copy(hv.at[src_v.at[S, 0]], rows_v.at[S], gsem[S])
            pltpu.async_copy(ev_hbm.at[pl.ds(r * (_W // 2), _W // 2)],
                             e_v.at[S], esem[S])

        def drain(sem, dst, dummy):
            pltpu.make_async_copy(dummy.at[pl.ds(0, dst.shape[0])], dst,
                                  sem).wait()

        for S in (0, 1):
            pltpu.sync_copy(src_hbm.at[pl.ds(lo + S, 1)], src_v.at[S])
            pltpu.sync_copy(dst_hbm.at[pl.ds(lo + S, 1)],
                            dst_v.at[S, pl.ds(0, 1)])
            launch(S, lo + S)

        def body(kk, carry):
            q = kk & 1
            for S in (0, 1):
                i = lo + 2 * kk + S
                drain(gsem[S], rows_v.at[S], hv)

                @pl.when(kk >= 1)
                def _drain_sc():
                    drain(ssem[S], m_v.at[S], hv)

                @pl.when(i + 2 < hi)
                def _idx_prefetch():
                    pltpu.async_copy(src_hbm.at[pl.ds(i + 2, 1)],
                                     src_v.at[S], isem[S])
                    pltpu.async_copy(dst_hbm.at[pl.ds(i + 2, 1)],
                                     dst_v.at[S, pl.ds(1 - q, 1)], isem[S])

                drain(esem[S], e_v.at[S], ev_hbm)

                def crow(pr, cy):
                    for half in range(2):
                        rr = 2 * pr + half
                        for j in range(_HH // 16):
                            sl = pl.ds(j * 16, 16)
                            esl = pl.ds(half * _HH + j * 16, 16)
                            m_v[S, rr, sl] = jnp.maximum(
                                rows_v[S, rr, sl] + e_v[S, pr, esl], 0.0)
                    return cy

                lax.fori_loop(0, _W // 2, crow, 0)
                pltpu.async_copy(m_v.at[S], agg_sh.at[dst_v.at[S, q]],
                                 ssem[S], add=True)

                @pl.when(i + 2 < hi)
                def _next_launch():
                    drain(isem[S], src_v.at[S], src_hbm)
                    drain(isem[S], dst_v.at[S, pl.ds(1 - q, 1)], src_hbm)
                    launch(S, i + 2)
            return carry

        lax.fori_loop(0, cnt // 2, body, 0)
        for S in (0, 1):
            drain(ssem[S], m_v.at[S], hv)
        plsc.subcore_barrier()

        for t in range(_STRIPE // _SCH):
            rr = r0 + t * _SCH
            pltpu.sync_copy(agg_sh.at[pl.ds(rr, _SCH)],
                            rows_v.at[0, pl.ds(0, _SCH)])
            pltpu.sync_copy(rows_v.at[0, pl.ds(0, _SCH)],
                            out_hbm.at[c, pl.ds(rr, _SCH)])

        @pl.when(s == 15)
        def _write_tail():
            pltpu.sync_copy(agg_sh.at[pl.ds(16 * _STRIPE, _TAIL)],
                            rows_v.at[0, pl.ds(0, _TAIL)])
            pltpu.sync_copy(rows_v.at[0, pl.ds(0, _TAIL)],
                            out_hbm.at[c, pl.ds(16 * _STRIPE, _TAIL)])

    return k(hp, ep, src2, dst2)


def kernel(x, edge_attr, params, edge_index, batch):
    src2 = edge_index[0].reshape(_NROW, _W)
    dst2 = edge_index[1].reshape(_NROW, _W)
    hp = _node_emb(x, params["node_emb"]["W"], params["node_emb"]["b"])
    h = jnp.concatenate([hp[0], hp[1]], axis=1)
    ea2 = edge_attr.reshape(_E // 2, 2 * edge_attr.shape[1])
    ep = _edge_emb(ea2, params["edge_emb"]["W"], params["edge_emb"]["b"])
    for li, lp in enumerate(params["layers"]):
        agg = _sc_message(hp, ep, src2, dst2)
        h = _mlp_bn(h, agg, lp)
        if li + 1 < len(params["layers"]):
            hp = jnp.stack([h[:, :_HH], h[:, _HH:]])
    return _pool_head(h, batch, params)


# speedup vs baseline: 1.5185x; 1.1076x over previous
import functools

import jax
import jax.numpy as jnp
from jax import lax
from jax.experimental import pallas as pl
from jax.experimental.pallas import tpu as pltpu
from jax.experimental.pallas import tpu_sc as plsc

_N = 10000
_E = 320000
_H = 128
_HH = _H // 2
_NG = 64
_MD = 16

_W = 128
_NROW = _E // _W
_STRIPE = 624
_SCH = 104
_TAIL = _N - 16 * _STRIPE


def _split_emb_body(x_ref, w_ref, b_ref, o_ref):
    o_ref[0] = (
        jnp.dot(x_ref[...], w_ref[0], preferred_element_type=jnp.float32)
        + b_ref[0]
    )


def _node_emb(x, w, b):
    w2 = jnp.stack([w[:, :_HH], w[:, _HH:]])
    b2 = jnp.stack([b[:_HH], b[_HH:]]).reshape(2, 1, _HH)
    return pl.pallas_call(
        _split_emb_body,
        grid=(2,),
        in_specs=[
            pl.BlockSpec((_N, x.shape[1]), lambda j: (0, 0)),
            pl.BlockSpec((1, x.shape[1], _HH), lambda j: (j, 0, 0)),
            pl.BlockSpec((1, 1, _HH), lambda j: (j, 0, 0)),
        ],
        out_specs=pl.BlockSpec((1, _N, _HH), lambda j: (j, 0, 0)),
        out_shape=jax.ShapeDtypeStruct((2, _N, _HH), jnp.float32),
    )(x, w2, b2)


def _edge_emb_body(alo_ref, ahi_ref, w_ref, b_ref, o_ref):
    alo = alo_ref[...]
    ahi = ahi_ref[...]
    for j in range(2):
        w = w_ref[j]
        b = b_ref[j]
        el = jnp.dot(alo, w, preferred_element_type=jnp.float32) + b
        er = jnp.dot(ahi, w, preferred_element_type=jnp.float32) + b
        o_ref[j] = jnp.concatenate([el, er], axis=1)


def _edge_emb(ea, w, b):
    blk = 10000
    de = ea.shape[1]
    nb = _E // 2 // blk
    w2 = jnp.stack([w[:, :_HH], w[:, _HH:]])
    b2 = jnp.stack([b[:_HH], b[_HH:]]).reshape(2, 1, _HH)
    return pl.pallas_call(
        _edge_emb_body,
        grid=(nb,),
        in_specs=[
            pl.BlockSpec((blk, de), lambda i: (i, 0)),
            pl.BlockSpec((blk, de), lambda i: (i + nb, 0)),
            pl.BlockSpec((2, de, _HH), lambda i: (0, 0, 0)),
            pl.BlockSpec((2, 1, _HH), lambda i: (0, 0, 0)),
        ],
        out_specs=pl.BlockSpec((2, blk, _H), lambda i: (0, i, 0)),
        out_shape=jax.ShapeDtypeStruct((2, _E // 2, _H), jnp.float32),
    )(ea, ea, w2, b2)


def _mlp_bn_body(h_ref, agg_ref, w1_ref, b1_ref, w2_ref, b2_ref, g_ref,
                 bb_ref, o_ref):
    z = h_ref[...] + jnp.concatenate([agg_ref[0], agg_ref[1]], axis=1)
    z1 = jnp.maximum(
        jnp.dot(z, w1_ref[...], preferred_element_type=jnp.float32)
        + b1_ref[...], 0.0)
    z2 = (jnp.dot(z1, w2_ref[...], preferred_element_type=jnp.float32)
          + b2_ref[...])
    mean = jnp.mean(z2, axis=0, keepdims=True)
    var = jnp.mean((z2 - mean) ** 2, axis=0, keepdims=True)
    zn = (z2 - mean) * lax.rsqrt(var + 1e-5) * g_ref[...] + bb_ref[...]
    o_ref[...] = jnp.maximum(zn, 0.0)


def _mlp_bn(h, agg, lp):
    return pl.pallas_call(
        _mlp_bn_body,
        out_shape=jax.ShapeDtypeStruct((_N, _H), jnp.float32),
    )(h, agg, lp["W1"], lp["b1"].reshape(1, -1), lp["W2"],
      lp["b2"].reshape(1, -1), lp["bn_g"].reshape(1, -1),
      lp["bn_b"].reshape(1, -1))


def _pool_head_body(h_ref, b_ref, wc_ref, bc_ref, wu_ref, bu_ref, wfu_ref,
                    wfc_ref, bf_ref, o_ref):
    gids = lax.broadcasted_iota(jnp.int32, (_N, _NG), 1)
    onehot = (b_ref[...] == gids).astype(jnp.float32)
    sums = lax.dot_general(onehot, h_ref[...], (((0,), (0,)), ((), ())),
                           preferred_element_type=jnp.float32)
    counts = jnp.sum(onehot, axis=0)[:, None]
    gx = sums / jnp.maximum(counts, 1.0)
    eu = jnp.dot(gx, wu_ref[...], preferred_element_type=jnp.float32) + bu_ref[...]
    ec = jnp.dot(gx, wc_ref[...], preferred_element_type=jnp.float32) + bc_ref[...]
    o_ref[...] = (
        jnp.dot(eu, wfu_ref[...], preferred_element_type=jnp.float32)
        + jnp.dot(ec, wfc_ref[...], preferred_element_type=jnp.float32)
        + bf_ref[...])


def _pool_head(h, batch, params):
    wf = params["final"]["W"]
    nc = wf.shape[1]
    return pl.pallas_call(
        _pool_head_body,
        out_shape=jax.ShapeDtypeStruct((_NG, nc), jnp.float32),
    )(h, batch.reshape(_N, 1),
      params["lin_common"]["W"], params["lin_common"]["b"].reshape(1, -1),
      params["lin_uncommon"]["W"], params["lin_uncommon"]["b"].reshape(1, -1),
      wf[:_MD], wf[_MD:], params["final"]["b"].reshape(1, -1))


def _sc_message(hp, ep, src2, dst2):
    mesh = plsc.VectorSubcoreMesh(core_axis_name="c", subcore_axis_name="s")

    @functools.partial(
        pl.kernel,
        mesh=mesh,
        compiler_params=pltpu.CompilerParams(use_tc_tiling_on_sc=False),
        out_type=jax.ShapeDtypeStruct((2, _N, _HH), jnp.float32),
        scratch_types=[
            pltpu.VMEM((2, 1, _W), jnp.int32),
            pltpu.VMEM((2, 2, _W), jnp.int32),
            pltpu.VMEM((2, _W, _HH), jnp.float32),
            pltpu.VMEM((2, _W, _HH), jnp.float32),
            pltpu.VMEM((2, _W, _HH), jnp.float32),
            pltpu.VMEM_SHARED((_N, _HH), jnp.float32),
            pltpu.SemaphoreType.DMA,
            pltpu.SemaphoreType.DMA,
            pltpu.SemaphoreType.DMA,
            pltpu.SemaphoreType.DMA,
            pltpu.SemaphoreType.DMA,
            pltpu.SemaphoreType.DMA,
            pltpu.SemaphoreType.DMA,
            pltpu.SemaphoreType.DMA,
        ],
    )
    def k(hp_hbm, ep_hbm, src_hbm, dst_hbm, out_hbm, src_v, dst_v, rows_v,
          e_v, m_v, agg_sh, gs0, gs1, es0, es1, ss0, ss1, is0, is1):
        c = lax.axis_index("c")
        s = lax.axis_index("s")
        gsem = (gs0, gs1)
        esem = (es0, es1)
        ssem = (ss0, ss1)
        isem = (is0, is1)
        hv = hp_hbm.at[c]
        ev_hbm = ep_hbm.at[c]

        zero16 = jnp.zeros((16,), jnp.float32)

        def zrow(r, carry):
            for j in range(_HH // 16):
                m_v[0, r, pl.ds(j * 16, 16)] = zero16
            return carry

        lax.fori_loop(0, _SCH, zrow, 0)
        r0 = s * _STRIPE
        for t in range(_STRIPE // _SCH):
            pltpu.sync_copy(m_v.at[0, pl.ds(0, _SCH)],
                            agg_sh.at[pl.ds(r0 + t * _SCH, _SCH)])

        @pl.when(s == 15)
        def _zero_tail():
            pltpu.sync_copy(m_v.at[0, pl.ds(0, _TAIL)],
                            agg_sh.at[pl.ds(16 * _STRIPE, _TAIL)])

        plsc.subcore_barrier()

        lo = 156 * s + 2 * jnp.maximum(s - 14, 0)
        cnt = 156 + 2 * (s >= 14).astype(jnp.int32)
        hi = lo + cnt

        def launch(S, r):
            pltpu.async_copy(hv.at[src_v.at[S, 0]], rows_v.at[S], gsem[S])
            ge = (r >= _NROW // 2).astype(jnp.int32)
            row0 = r * _W - (_E // 2) * ge
            pltpu.async_copy(
                ev_hbm.at[pl.ds(row0, _W), pl.ds(ge * _HH, _HH)],
                e_v.at[S], esem[S])

        def drain(sem, dst, dummy):
            pltpu.make_async_copy(dummy.at[pl.ds(0, dst.shape[0])], dst,
                                  sem).wait()

        for S in (0, 1):
            pltpu.sync_copy(src_hbm.at[pl.ds(lo + S, 1)], src_v.at[S])
            pltpu.sync_copy(dst_hbm.at[pl.ds(lo + S, 1)],
                            dst_v.at[S, pl.ds(0, 1)])
            launch(S, lo + S)

        def body(kk, carry):
            q = kk & 1
            for S in (0, 1):
                i = lo + 2 * kk + S
                drain(gsem[S], rows_v.at[S], hv)

                @pl.when(kk >= 1)
                def _drain_sc():
                    drain(ssem[S], m_v.at[S], hv)

                @pl.when(i + 2 < hi)
                def _idx_prefetch():
                    pltpu.async_copy(src_hbm.at[pl.ds(i + 2, 1)],
                                     src_v.at[S], isem[S])
                    pltpu.async_copy(dst_hbm.at[pl.ds(i + 2, 1)],
                                     dst_v.at[S, pl.ds(1 - q, 1)], isem[S])

                drain(esem[S], e_v.at[S], hv)

                def crow(rr, cy):
                    for j in range(_HH // 16):
                        sl = pl.ds(j * 16, 16)
                        m_v[S, rr, sl] = jnp.maximum(
                            rows_v[S, rr, sl] + e_v[S, rr, sl], 0.0)
                    return cy

                lax.fori_loop(0, _W, crow, 0)
                pltpu.async_copy(m_v.at[S], agg_sh.at[dst_v.at[S, q]],
                                 ssem[S], add=True)

                @pl.when(i + 2 < hi)
                def _next_launch():
                    drain(isem[S], src_v.at[S], src_hbm)
                    drain(isem[S], dst_v.at[S, pl.ds(1 - q, 1)], src_hbm)
                    launch(S, i + 2)
            return carry

        lax.fori_loop(0, cnt // 2, body, 0)
        for S in (0, 1):
            drain(ssem[S], m_v.at[S], hv)
        plsc.subcore_barrier()

        for t in range(_STRIPE // _SCH):
            rr = r0 + t * _SCH
            pltpu.sync_copy(agg_sh.at[pl.ds(rr, _SCH)],
                            rows_v.at[0, pl.ds(0, _SCH)])
            pltpu.sync_copy(rows_v.at[0, pl.ds(0, _SCH)],
                            out_hbm.at[c, pl.ds(rr, _SCH)])

        @pl.when(s == 15)
        def _write_tail():
            pltpu.sync_copy(agg_sh.at[pl.ds(16 * _STRIPE, _TAIL)],
                            rows_v.at[0, pl.ds(0, _TAIL)])
            pltpu.sync_copy(rows_v.at[0, pl.ds(0, _TAIL)],
                            out_hbm.at[c, pl.ds(16 * _STRIPE, _TAIL)])

    return k(hp, ep, src2, dst2)


def kernel(x, edge_attr, params, edge_index, batch):
    src2 = edge_index[0].reshape(_NROW, _W)
    dst2 = edge_index[1].reshape(_NROW, _W)
    hp = _node_emb(x, params["node_emb"]["W"], params["node_emb"]["b"])
    h = jnp.concatenate([hp[0], hp[1]], axis=1)
    ep = _edge_emb(edge_attr, params["edge_emb"]["W"], params["edge_emb"]["b"])
    for li, lp in enumerate(params["layers"]):
        agg = _sc_message(hp, ep, src2, dst2)
        h = _mlp_bn(h, agg, lp)
        if li + 1 < len(params["layers"]):
            hp = jnp.stack([h[:, :_HH], h[:, _HH:]])
    return _pool_head(h, batch, params)
